# baseline (speedup 1.0000x reference)
import functools

import jax
import jax.numpy as jnp
from jax import lax
from jax.experimental import pallas as pl
from jax.experimental.pallas import tpu as pltpu
from jax.experimental.pallas import tpu_sc as plsc

N = 10000
K = 16
D = 128
DA = 160
E = 320000
NB = 10
BR = N // NB
NC = 2
NS = 16
NW = NC * NS
CH = 280
NCH = 36
FAST_CORE = 1
SPLIT = 22
EWP = NCH * CH
EP = NW * EWP
NP = 10240
F32 = jnp.float32
BF16 = jnp.bfloat16


def _prep_body(cw_ref, emb_ref, h0_ref, h1_ref, h2_ref,
               table_ref, s1_ref, s2_ref, aux_ref):
    i = pl.program_id(0)
    cw = cw_ref[...]
    emb = emb_ref[...]
    nrm = jnp.sqrt(jnp.sum(emb * emb, axis=1, keepdims=True))
    normed = emb / jnp.maximum(nrm, 1e-8)
    table_ref[...] = jnp.concatenate(
        [normed, jnp.ones((BR, 1), F32), jnp.zeros((BR, DA - D - 1), F32)],
        axis=1).astype(BF16)

    @pl.when(i == 0)
    def _():
        s1_ref[...] = jnp.zeros_like(s1_ref)
        s2_ref[...] = jnp.zeros_like(s2_ref)
        aux_ref[...] = jnp.zeros_like(aux_ref)

    cw2 = cw * cw
    dn = (((0,), (0,)), ((), ()))
    m1 = []
    m2 = []
    nsum = []
    for f_ref in (h0_ref, h1_ref, h2_ref):
        feat = f_ref[...]
        m1.append(lax.dot_general(cw, feat, dn, preferred_element_type=F32))
        m2.append(lax.dot_general(cw2, feat * feat, dn,
                                  preferred_element_type=F32))
        nsum.append(jnp.sum(jnp.sqrt(jnp.sum(feat * feat, axis=1))))
    s1_ref[...] += jnp.concatenate(m1, axis=0)
    s2_ref[...] += jnp.concatenate(m2, axis=0)

    csum = jnp.sum(cw, axis=0, keepdims=True)
    row0 = jnp.concatenate([csum, jnp.zeros((1, D - K), F32)], axis=1)
    lane = lax.broadcasted_iota(jnp.int32, (1, D), 1)
    row1 = (jnp.where(lane == 0, nsum[0], 0.0)
            + jnp.where(lane == 1, nsum[1], 0.0)
            + jnp.where(lane == 2, nsum[2], 0.0)).astype(F32)
    aux_ref[...] += jnp.concatenate(
        [row0, row1, jnp.zeros((6, D), F32)], axis=0)


_prep_call = pl.pallas_call(
    _prep_body,
    grid=(NB,),
    in_specs=[
        pl.BlockSpec((BR, K), lambda i: (i, 0)),
        pl.BlockSpec((BR, D), lambda i: (i, 0)),
        pl.BlockSpec((BR, D), lambda i: (i, 0)),
        pl.BlockSpec((BR, D), lambda i: (i, 0)),
        pl.BlockSpec((BR, D), lambda i: (i, 0)),
    ],
    out_specs=[
        pl.BlockSpec((BR, DA), lambda i: (i, 0)),
        pl.BlockSpec((3 * K, D), lambda i: (0, 0)),
        pl.BlockSpec((3 * K, D), lambda i: (0, 0)),
        pl.BlockSpec((8, D), lambda i: (0, 0)),
    ],
    out_shape=[
        jax.ShapeDtypeStruct((N, DA), BF16),
        jax.ShapeDtypeStruct((3 * K, D), F32),
        jax.ShapeDtypeStruct((3 * K, D), F32),
        jax.ShapeDtypeStruct((8, D), F32),
    ],
)


def _edge_body(row_hbm, col_hbm, table_hbm, zeros_hbm, out_hbm,
               row_v, col_v, prow_v, pcol_v, rows0, rows1, acc_sh,
               gsem0, gsem1):
    c = lax.axis_index("c")
    s = lax.axis_index("s")
    wid = s * NC + c
    pwid = s * NC + (1 - c)
    rps = NP // NS
    pltpu.sync_copy(zeros_hbm, acc_sh.at[pl.ds(s * rps, rps)])

    pltpu.sync_copy(row_hbm.at[wid], row_v)
    pltpu.sync_copy(col_hbm.at[wid], col_v)

    @pl.when(c == FAST_CORE)
    def _():
        pltpu.sync_copy(row_hbm.at[pwid, pl.ds(SPLIT, NCH - SPLIT)], prow_v)
        pltpu.sync_copy(col_hbm.at[pwid, pl.ds(SPLIT, NCH - SPLIT)], pcol_v)

    plsc.subcore_barrier()

    bufs = (rows0, rows1)
    gsems = (gsem0, gsem1)

    def run_ring(rv, cv, nch):
        def gather_start(g, b):
            pltpu.async_copy(table_hbm.at[cv.at[g]], bufs[b], gsems[b])

        def gather_wait(g, b):
            pltpu.make_async_copy(table_hbm.at[cv.at[g]], bufs[b],
                                  gsems[b]).wait()

        def scatter(g, b):
            pltpu.sync_copy(bufs[b], acc_sh.at[rv.at[g]], add=True)

        gather_start(0, 0)
        gather_start(1, 1)

        def step(i, carry):
            for b in range(2):
                g = 2 * i + b
                gather_wait(g, b)
                scatter(g, b)
                gather_start(g + 2, b)
            return carry

        lax.fori_loop(0, nch // 2 - 1, step, 0)
        for b in range(2):
            gather_wait(nch - 2 + b, b)
            scatter(nch - 2 + b, b)

    @pl.when(c == FAST_CORE)
    def _():
        run_ring(row_v, col_v, NCH)
        run_ring(prow_v, pcol_v, NCH - SPLIT)

    @pl.when(c != FAST_CORE)
    def _():
        run_ring(row_v, col_v, SPLIT)

    plsc.subcore_barrier()
    pltpu.sync_copy(acc_sh.at[pl.ds(s * rps, rps)],
                    out_hbm.at[c, pl.ds(s * rps, rps)])


@functools.cache
def _edge_call():
    return functools.partial(
        pl.kernel,
        out_type=jax.ShapeDtypeStruct((NC, NP, DA), BF16),
        mesh=plsc.VectorSubcoreMesh(core_axis_name="c", subcore_axis_name="s",
                                    num_cores=NC, num_subcores=NS),
        scratch_types=[
            pltpu.VMEM((NCH, CH), jnp.int32),
            pltpu.VMEM((NCH, CH), jnp.int32),
            pltpu.VMEM((NCH - SPLIT, CH), jnp.int32),
            pltpu.VMEM((NCH - SPLIT, CH), jnp.int32),
            pltpu.VMEM((CH, DA), BF16),
            pltpu.VMEM((CH, DA), BF16),
            pltpu.VMEM_SHARED((NP, DA), BF16),
            pltpu.SemaphoreType.DMA,
            pltpu.SemaphoreType.DMA,
        ],
        compiler_params=pltpu.CompilerParams(use_tc_tiling_on_sc=False),
    )(_edge_body)


def _combine_body(parts_ref, table_ref, cw_ref, s1_ref, s2_ref, aux_ref,
                  out_ref, nacc_ref):
    i = pl.program_id(0)

    @pl.when(i == 0)
    def _():
        nacc_ref[...] = jnp.zeros_like(nacc_ref)

    p = parts_ref[...].astype(F32)
    ssum = p[0] + p[1]
    sv = ssum[:, :D]
    deg = ssum[:, D:D + 1] + 1e-8
    normed = table_ref[:, :D].astype(F32)
    nc = jnp.sum(normed * sv, axis=1, keepdims=True) / deg
    dn = (((0,), (0,)), ((), ()))
    nacc_ref[...] += lax.dot_general(nc, cw_ref[...], dn,
                                     preferred_element_type=F32)

    @pl.when(i == NB - 1)
    def _():
        s1 = s1_ref[...]
        s2 = s2_ref[...]
        var = (s2 - s1 * s1 * (1.0 / N)) * (1.0 / (N - 1))
        vmean = jnp.mean(var, axis=1, keepdims=True)
        w = jnp.concatenate([jnp.full((K, 1), 1.0, F32),
                             jnp.full((K, 1), 0.5, F32),
                             jnp.full((K, 1), 0.25, F32)], axis=0)
        hop_loss = jnp.sum(w / (vmean + 1e-8)) / K
        aux = aux_ref[...]
        csum = aux[0:1, :K]
        congestion = jnp.sum(nacc_ref[...] / (csum + 1e-8)) / K
        m0 = aux[1, 0] / N
        m1 = aux[1, 1] / N
        m2 = aux[1, 2] / N
        flow = jnp.maximum(m1 - m0, 0.0) + jnp.maximum(m2 - m1, 0.0)
        total = hop_loss + 0.5 * congestion + flow
        out_ref[...] = jnp.broadcast_to(total, (1, 1)).astype(F32)


_combine_call = pl.pallas_call(
    _combine_body,
    grid=(NB,),
    in_specs=[
        pl.BlockSpec((NC, BR, DA), lambda i: (0, i, 0)),
        pl.BlockSpec((BR, DA), lambda i: (i, 0)),
        pl.BlockSpec((BR, K), lambda i: (i, 0)),
        pl.BlockSpec((3 * K, D), lambda i: (0, 0)),
        pl.BlockSpec((3 * K, D), lambda i: (0, 0)),
        pl.BlockSpec((8, D), lambda i: (0, 0)),
    ],
    out_specs=pl.BlockSpec((1, 1), lambda i: (0, 0)),
    out_shape=jax.ShapeDtypeStruct((1, 1), F32),
    scratch_shapes=[pltpu.VMEM((1, K), F32)],
)


@jax.jit
def kernel(cluster_assignments, network_embeddings, hop_0_features,
           hop_1_features, hop_2_features, edge_index):
    table, s1, s2, aux = _prep_call(
        cluster_assignments, network_embeddings,
        hop_0_features, hop_1_features, hop_2_features)
    zeros = jnp.zeros((NP // NS, DA), BF16)
    pad = EP - E
    row3 = jnp.concatenate(
        [edge_index[0],
         N + (jnp.arange(pad, dtype=jnp.int32) % (NP - N))]
    ).reshape(NW, NCH, CH)
    col3 = jnp.concatenate(
        [edge_index[1], jnp.zeros((pad,), jnp.int32)]).reshape(NW, NCH, CH)
    parts = _edge_call()(row3, col3, table, zeros)
    total = _combine_call(parts, table, cluster_assignments, s1, s2, aux)
    return total[0, 0]


# speedup vs baseline: 1.0070x; 1.0070x over previous
import functools

import jax
import jax.numpy as jnp
from jax import lax
from jax.experimental import pallas as pl
from jax.experimental.pallas import tpu as pltpu
from jax.experimental.pallas import tpu_sc as plsc

N = 10000
K = 16
D = 128
DA = 160
E = 320000
NB = 10
BR = N // NB
NC = 2
NS = 16
NW = NC * NS
CH = 280
NCH = 36
FAST_CORE = 0
SPLIT = 22
EWP = NCH * CH
EP = NW * EWP
NP = 10240
F32 = jnp.float32
BF16 = jnp.bfloat16


def _prep_body(cw_ref, emb_ref, h0_ref, h1_ref, h2_ref,
               table_ref, s1_ref, s2_ref, aux_ref):
    i = pl.program_id(0)
    cw = cw_ref[...]
    emb = emb_ref[...]
    nrm = jnp.sqrt(jnp.sum(emb * emb, axis=1, keepdims=True))
    normed = emb / jnp.maximum(nrm, 1e-8)
    table_ref[...] = jnp.concatenate(
        [normed, jnp.ones((BR, 1), F32), jnp.zeros((BR, DA - D - 1), F32)],
        axis=1).astype(BF16)

    @pl.when(i == 0)
    def _():
        s1_ref[...] = jnp.zeros_like(s1_ref)
        s2_ref[...] = jnp.zeros_like(s2_ref)
        aux_ref[...] = jnp.zeros_like(aux_ref)

    cw2 = cw * cw
    dn = (((0,), (0,)), ((), ()))
    m1 = []
    m2 = []
    nsum = []
    for f_ref in (h0_ref, h1_ref, h2_ref):
        feat = f_ref[...]
        m1.append(lax.dot_general(cw, feat, dn, preferred_element_type=F32))
        m2.append(lax.dot_general(cw2, feat * feat, dn,
                                  preferred_element_type=F32))
        nsum.append(jnp.sum(jnp.sqrt(jnp.sum(feat * feat, axis=1))))
    s1_ref[...] += jnp.concatenate(m1, axis=0)
    s2_ref[...] += jnp.concatenate(m2, axis=0)

    csum = jnp.sum(cw, axis=0, keepdims=True)
    row0 = jnp.concatenate([csum, jnp.zeros((1, D - K), F32)], axis=1)
    lane = lax.broadcasted_iota(jnp.int32, (1, D), 1)
    row1 = (jnp.where(lane == 0, nsum[0], 0.0)
            + jnp.where(lane == 1, nsum[1], 0.0)
            + jnp.where(lane == 2, nsum[2], 0.0)).astype(F32)
    aux_ref[...] += jnp.concatenate(
        [row0, row1, jnp.zeros((6, D), F32)], axis=0)


_prep_call = pl.pallas_call(
    _prep_body,
    grid=(NB,),
    in_specs=[
        pl.BlockSpec((BR, K), lambda i: (i, 0)),
        pl.BlockSpec((BR, D), lambda i: (i, 0)),
        pl.BlockSpec((BR, D), lambda i: (i, 0)),
        pl.BlockSpec((BR, D), lambda i: (i, 0)),
        pl.BlockSpec((BR, D), lambda i: (i, 0)),
    ],
    out_specs=[
        pl.BlockSpec((BR, DA), lambda i: (i, 0)),
        pl.BlockSpec((3 * K, D), lambda i: (0, 0)),
        pl.BlockSpec((3 * K, D), lambda i: (0, 0)),
        pl.BlockSpec((8, D), lambda i: (0, 0)),
    ],
    out_shape=[
        jax.ShapeDtypeStruct((N, DA), BF16),
        jax.ShapeDtypeStruct((3 * K, D), F32),
        jax.ShapeDtypeStruct((3 * K, D), F32),
        jax.ShapeDtypeStruct((8, D), F32),
    ],
)


def _edge_body(row_hbm, col_hbm, table_hbm, zeros_hbm, out_hbm,
               row_v, col_v, prow_v, pcol_v, rows0, rows1, acc_sh,
               gsem0, gsem1):
    c = lax.axis_index("c")
    s = lax.axis_index("s")
    wid = s * NC + c
    pwid = s * NC + (1 - c)
    rps = NP // NS
    pltpu.sync_copy(zeros_hbm, acc_sh.at[pl.ds(s * rps, rps)])

    pltpu.sync_copy(row_hbm.at[wid], row_v)
    pltpu.sync_copy(col_hbm.at[wid], col_v)

    @pl.when(c == FAST_CORE)
    def _():
        pltpu.sync_copy(row_hbm.at[pwid, pl.ds(SPLIT, NCH - SPLIT)], prow_v)
        pltpu.sync_copy(col_hbm.at[pwid, pl.ds(SPLIT, NCH - SPLIT)], pcol_v)

    plsc.subcore_barrier()

    bufs = (rows0, rows1)
    gsems = (gsem0, gsem1)

    def run_ring(rv, cv, nch):
        def gather_start(g, b):
            pltpu.async_copy(table_hbm.at[cv.at[g]], bufs[b], gsems[b])

        def gather_wait(g, b):
            pltpu.make_async_copy(table_hbm.at[cv.at[g]], bufs[b],
                                  gsems[b]).wait()

        def scatter(g, b):
            pltpu.sync_copy(bufs[b], acc_sh.at[rv.at[g]], add=True)

        gather_start(0, 0)
        gather_start(1, 1)

        def step(i, carry):
            for b in range(2):
                g = 2 * i + b
                gather_wait(g, b)
                scatter(g, b)
                gather_start(g + 2, b)
            return carry

        lax.fori_loop(0, nch // 2 - 1, step, 0)
        for b in range(2):
            gather_wait(nch - 2 + b, b)
            scatter(nch - 2 + b, b)

    @pl.when(c == FAST_CORE)
    def _():
        run_ring(row_v, col_v, NCH)
        run_ring(prow_v, pcol_v, NCH - SPLIT)

    @pl.when(c != FAST_CORE)
    def _():
        run_ring(row_v, col_v, SPLIT)

    plsc.subcore_barrier()
    pltpu.sync_copy(acc_sh.at[pl.ds(s * rps, rps)],
                    out_hbm.at[c, pl.ds(s * rps, rps)])


@functools.cache
def _edge_call():
    return functools.partial(
        pl.kernel,
        out_type=jax.ShapeDtypeStruct((NC, NP, DA), BF16),
        mesh=plsc.VectorSubcoreMesh(core_axis_name="c", subcore_axis_name="s",
                                    num_cores=NC, num_subcores=NS),
        scratch_types=[
            pltpu.VMEM((NCH, CH), jnp.int32),
            pltpu.VMEM((NCH, CH), jnp.int32),
            pltpu.VMEM((NCH - SPLIT, CH), jnp.int32),
            pltpu.VMEM((NCH - SPLIT, CH), jnp.int32),
            pltpu.VMEM((CH, DA), BF16),
            pltpu.VMEM((CH, DA), BF16),
            pltpu.VMEM_SHARED((NP, DA), BF16),
            pltpu.SemaphoreType.DMA,
            pltpu.SemaphoreType.DMA,
        ],
        compiler_params=pltpu.CompilerParams(use_tc_tiling_on_sc=False),
    )(_edge_body)


def _combine_body(parts_ref, table_ref, cw_ref, s1_ref, s2_ref, aux_ref,
                  out_ref, nacc_ref):
    i = pl.program_id(0)

    @pl.when(i == 0)
    def _():
        nacc_ref[...] = jnp.zeros_like(nacc_ref)

    p = parts_ref[...].astype(F32)
    ssum = p[0] + p[1]
    sv = ssum[:, :D]
    deg = ssum[:, D:D + 1] + 1e-8
    normed = table_ref[:, :D].astype(F32)
    nc = jnp.sum(normed * sv, axis=1, keepdims=True) / deg
    dn = (((0,), (0,)), ((), ()))
    nacc_ref[...] += lax.dot_general(nc, cw_ref[...], dn,
                                     preferred_element_type=F32)

    @pl.when(i == NB - 1)
    def _():
        s1 = s1_ref[...]
        s2 = s2_ref[...]
        var = (s2 - s1 * s1 * (1.0 / N)) * (1.0 / (N - 1))
        vmean = jnp.mean(var, axis=1, keepdims=True)
        w = jnp.concatenate([jnp.full((K, 1), 1.0, F32),
                             jnp.full((K, 1), 0.5, F32),
                             jnp.full((K, 1), 0.25, F32)], axis=0)
        hop_loss = jnp.sum(w / (vmean + 1e-8)) / K
        aux = aux_ref[...]
        csum = aux[0:1, :K]
        congestion = jnp.sum(nacc_ref[...] / (csum + 1e-8)) / K
        m0 = aux[1, 0] / N
        m1 = aux[1, 1] / N
        m2 = aux[1, 2] / N
        flow = jnp.maximum(m1 - m0, 0.0) + jnp.maximum(m2 - m1, 0.0)
        total = hop_loss + 0.5 * congestion + flow
        out_ref[...] = jnp.broadcast_to(total, (1, 1)).astype(F32)


_combine_call = pl.pallas_call(
    _combine_body,
    grid=(NB,),
    in_specs=[
        pl.BlockSpec((NC, BR, DA), lambda i: (0, i, 0)),
        pl.BlockSpec((BR, DA), lambda i: (i, 0)),
        pl.BlockSpec((BR, K), lambda i: (i, 0)),
        pl.BlockSpec((3 * K, D), lambda i: (0, 0)),
        pl.BlockSpec((3 * K, D), lambda i: (0, 0)),
        pl.BlockSpec((8, D), lambda i: (0, 0)),
    ],
    out_specs=pl.BlockSpec((1, 1), lambda i: (0, 0)),
    out_shape=jax.ShapeDtypeStruct((1, 1), F32),
    scratch_shapes=[pltpu.VMEM((1, K), F32)],
)


@jax.jit
def kernel(cluster_assignments, network_embeddings, hop_0_features,
           hop_1_features, hop_2_features, edge_index):
    table, s1, s2, aux = _prep_call(
        cluster_assignments, network_embeddings,
        hop_0_features, hop_1_features, hop_2_features)
    zeros = jnp.zeros((NP // NS, DA), BF16)
    pad = EP - E
    row3 = jnp.concatenate(
        [edge_index[0],
         N + (jnp.arange(pad, dtype=jnp.int32) % (NP - N))]
    ).reshape(NW, NCH, CH)
    col3 = jnp.concatenate(
        [edge_index[1], jnp.zeros((pad,), jnp.int32)]).reshape(NW, NCH, CH)
    parts = _edge_call()(row3, col3, table, zeros)
    total = _combine_call(parts, table, cluster_assignments, s1, s2, aux)
    return total[0, 0]


# speedup vs baseline: 1.4775x; 1.4673x over previous
import functools

import jax
import jax.numpy as jnp
from jax import lax
from jax.experimental import pallas as pl
from jax.experimental.pallas import tpu as pltpu
from jax.experimental.pallas import tpu_sc as plsc

N = 10000
K = 16
D = 128
DA = 160
E = 320000
NB = 10
BR = N // NB
NC = 2
NS = 16
NW = NC * NS
EW = E // NW
CH = 200
NCH = EW // CH
NP = 10240
F32 = jnp.float32
BF16 = jnp.bfloat16


def _prep_body(cw_ref, emb_ref, h0_ref, h1_ref, h2_ref,
               table_ref, s1_ref, s2_ref, aux_ref):
    i = pl.program_id(0)
    cw = cw_ref[...]
    emb = emb_ref[...]
    nrm = jnp.sqrt(jnp.sum(emb * emb, axis=1, keepdims=True))
    normed = emb / jnp.maximum(nrm, 1e-8)
    table_ref[...] = jnp.concatenate(
        [normed, jnp.ones((BR, 1), F32), jnp.zeros((BR, DA - D - 1), F32)],
        axis=1).astype(BF16)

    @pl.when(i == 0)
    def _():
        s1_ref[...] = jnp.zeros_like(s1_ref)
        s2_ref[...] = jnp.zeros_like(s2_ref)
        aux_ref[...] = jnp.zeros_like(aux_ref)

    cw2 = cw * cw
    dn = (((0,), (0,)), ((), ()))
    m1 = []
    m2 = []
    nsum = []
    for f_ref in (h0_ref, h1_ref, h2_ref):
        feat = f_ref[...]
        m1.append(lax.dot_general(cw, feat, dn, preferred_element_type=F32))
        m2.append(lax.dot_general(cw2, feat * feat, dn,
                                  preferred_element_type=F32))
        nsum.append(jnp.sum(jnp.sqrt(jnp.sum(feat * feat, axis=1))))
    s1_ref[...] += jnp.concatenate(m1, axis=0)
    s2_ref[...] += jnp.concatenate(m2, axis=0)

    csum = jnp.sum(cw, axis=0, keepdims=True)
    row0 = jnp.concatenate([csum, jnp.zeros((1, D - K), F32)], axis=1)
    lane = lax.broadcasted_iota(jnp.int32, (1, D), 1)
    row1 = (jnp.where(lane == 0, nsum[0], 0.0)
            + jnp.where(lane == 1, nsum[1], 0.0)
            + jnp.where(lane == 2, nsum[2], 0.0)).astype(F32)
    aux_ref[...] += jnp.concatenate(
        [row0, row1, jnp.zeros((6, D), F32)], axis=0)


_prep_call = pl.pallas_call(
    _prep_body,
    grid=(NB,),
    in_specs=[
        pl.BlockSpec((BR, K), lambda i: (i, 0)),
        pl.BlockSpec((BR, D), lambda i: (i, 0)),
        pl.BlockSpec((BR, D), lambda i: (i, 0)),
        pl.BlockSpec((BR, D), lambda i: (i, 0)),
        pl.BlockSpec((BR, D), lambda i: (i, 0)),
    ],
    out_specs=[
        pl.BlockSpec((BR, DA), lambda i: (i, 0)),
        pl.BlockSpec((3 * K, D), lambda i: (0, 0)),
        pl.BlockSpec((3 * K, D), lambda i: (0, 0)),
        pl.BlockSpec((8, D), lambda i: (0, 0)),
    ],
    out_shape=[
        jax.ShapeDtypeStruct((N, DA), BF16),
        jax.ShapeDtypeStruct((3 * K, D), F32),
        jax.ShapeDtypeStruct((3 * K, D), F32),
        jax.ShapeDtypeStruct((8, D), F32),
    ],
)


def _edge_body(row_hbm, col_hbm, table_hbm, zeros_hbm, out_hbm,
               row_v, col_v, rows0, rows1, acc_sh, gsem0, gsem1):
    c = lax.axis_index("c")
    s = lax.axis_index("s")
    wid = s * NC + c
    rps = NP // NS
    pltpu.sync_copy(zeros_hbm, acc_sh.at[pl.ds(s * rps, rps)])

    pltpu.sync_copy(row_hbm.at[pl.ds(wid * EW, EW)], row_v)
    pltpu.sync_copy(col_hbm.at[pl.ds(wid * EW, EW)], col_v)
    plsc.subcore_barrier()

    bufs = (rows0, rows1)
    gsems = (gsem0, gsem1)

    def cidx(ref, g):
        return ref.at[pl.ds(g * CH, CH)]

    def gather_start(g, b):
        pltpu.async_copy(table_hbm.at[cidx(col_v, g)], bufs[b], gsems[b])

    def gather_wait(g, b):
        pltpu.make_async_copy(table_hbm.at[cidx(col_v, g)], bufs[b],
                              gsems[b]).wait()

    def scatter(g, b):
        pltpu.sync_copy(bufs[b], acc_sh.at[cidx(row_v, g)], add=True)

    gather_start(0, 0)
    gather_start(1, 1)

    def step(i, carry):
        for b in range(2):
            g = 2 * i + b
            gather_wait(g, b)
            scatter(g, b)
            gather_start(g + 2, b)
        return carry

    lax.fori_loop(0, NCH // 2 - 1, step, 0)
    for b in range(2):
        gather_wait(NCH - 2 + b, b)
        scatter(NCH - 2 + b, b)

    plsc.subcore_barrier()
    pltpu.sync_copy(acc_sh.at[pl.ds(s * rps, rps)],
                    out_hbm.at[c, pl.ds(s * rps, rps)])


@functools.cache
def _edge_call():
    return functools.partial(
        pl.kernel,
        out_type=jax.ShapeDtypeStruct((NC, NP, DA), BF16),
        mesh=plsc.VectorSubcoreMesh(core_axis_name="c", subcore_axis_name="s",
                                    num_cores=NC, num_subcores=NS),
        scratch_types=[
            pltpu.VMEM((EW,), jnp.int32),
            pltpu.VMEM((EW,), jnp.int32),
            pltpu.VMEM((CH, DA), BF16),
            pltpu.VMEM((CH, DA), BF16),
            pltpu.VMEM_SHARED((NP, DA), BF16),
            pltpu.SemaphoreType.DMA,
            pltpu.SemaphoreType.DMA,
        ],
        compiler_params=pltpu.CompilerParams(use_tc_tiling_on_sc=False),
    )(_edge_body)


def _combine_body(parts_ref, table_ref, cw_ref, s1_ref, s2_ref, aux_ref,
                  out_ref, nacc_ref):
    i = pl.program_id(0)

    @pl.when(i == 0)
    def _():
        nacc_ref[...] = jnp.zeros_like(nacc_ref)

    p = parts_ref[...].astype(F32)
    ssum = p[0] + p[1]
    sv = ssum[:, :D]
    deg = ssum[:, D:D + 1] + 1e-8
    normed = table_ref[:, :D].astype(F32)
    nc = jnp.sum(normed * sv, axis=1, keepdims=True) / deg
    dn = (((0,), (0,)), ((), ()))
    nacc_ref[...] += lax.dot_general(nc, cw_ref[...], dn,
                                     preferred_element_type=F32)

    @pl.when(i == NB - 1)
    def _():
        s1 = s1_ref[...]
        s2 = s2_ref[...]
        var = (s2 - s1 * s1 * (1.0 / N)) * (1.0 / (N - 1))
        vmean = jnp.mean(var, axis=1, keepdims=True)
        w = jnp.concatenate([jnp.full((K, 1), 1.0, F32),
                             jnp.full((K, 1), 0.5, F32),
                             jnp.full((K, 1), 0.25, F32)], axis=0)
        hop_loss = jnp.sum(w / (vmean + 1e-8)) / K
        aux = aux_ref[...]
        csum = aux[0:1, :K]
        congestion = jnp.sum(nacc_ref[...] / (csum + 1e-8)) / K
        m0 = aux[1, 0] / N
        m1 = aux[1, 1] / N
        m2 = aux[1, 2] / N
        flow = jnp.maximum(m1 - m0, 0.0) + jnp.maximum(m2 - m1, 0.0)
        total = hop_loss + 0.5 * congestion + flow
        out_ref[...] = jnp.broadcast_to(total, (1, 1)).astype(F32)


_combine_call = pl.pallas_call(
    _combine_body,
    grid=(NB,),
    in_specs=[
        pl.BlockSpec((NC, BR, DA), lambda i: (0, i, 0)),
        pl.BlockSpec((BR, DA), lambda i: (i, 0)),
        pl.BlockSpec((BR, K), lambda i: (i, 0)),
        pl.BlockSpec((3 * K, D), lambda i: (0, 0)),
        pl.BlockSpec((3 * K, D), lambda i: (0, 0)),
        pl.BlockSpec((8, D), lambda i: (0, 0)),
    ],
    out_specs=pl.BlockSpec((1, 1), lambda i: (0, 0)),
    out_shape=jax.ShapeDtypeStruct((1, 1), F32),
    scratch_shapes=[pltpu.VMEM((1, K), F32)],
)


@jax.jit
def kernel(cluster_assignments, network_embeddings, hop_0_features,
           hop_1_features, hop_2_features, edge_index):
    table, s1, s2, aux = _prep_call(
        cluster_assignments, network_embeddings,
        hop_0_features, hop_1_features, hop_2_features)
    zeros = jnp.zeros((NP // NS, DA), BF16)
    parts = _edge_call()(edge_index[0], edge_index[1], table, zeros)
    total = _combine_call(parts, table, cluster_assignments, s1, s2, aux)
    return total[0, 0]


# speedup vs baseline: 1.4954x; 1.0121x over previous
import functools

import jax
import jax.numpy as jnp
from jax import lax
from jax.experimental import pallas as pl
from jax.experimental.pallas import tpu as pltpu
from jax.experimental.pallas import tpu_sc as plsc

N = 10000
K = 16
D = 128
DA = 160
E = 320000
NB = 10
BR = N // NB
NBA = 5
BRA = N // NBA
NC = 2
NS = 16
NW = NC * NS
EW = E // NW
CH = 200
NCH = EW // CH
NP = 10240
F32 = jnp.float32
BF16 = jnp.bfloat16


def _prep_body(cw_ref, emb_ref, h0_ref, h1_ref, h2_ref,
               table_ref, s1_ref, s2_ref, aux_ref):
    i = pl.program_id(0)
    cw = cw_ref[...]
    emb = emb_ref[...]
    nrm = jnp.sqrt(jnp.sum(emb * emb, axis=1, keepdims=True))
    normed = emb / jnp.maximum(nrm, 1e-8)
    table_ref[...] = jnp.concatenate(
        [normed, jnp.ones((BRA, 1), F32), jnp.zeros((BRA, DA - D - 1), F32)],
        axis=1).astype(BF16)

    @pl.when(i == 0)
    def _():
        s1_ref[...] = jnp.zeros_like(s1_ref)
        s2_ref[...] = jnp.zeros_like(s2_ref)
        aux_ref[...] = jnp.zeros_like(aux_ref)

    cw2 = cw * cw
    dn = (((0,), (0,)), ((), ()))
    m1 = []
    m2 = []
    nsum = []
    for f_ref in (h0_ref, h1_ref, h2_ref):
        feat = f_ref[...]
        m1.append(lax.dot_general(cw, feat, dn, preferred_element_type=F32))
        m2.append(lax.dot_general(cw2, feat * feat, dn,
                                  preferred_element_type=F32))
        nsum.append(jnp.sum(jnp.sqrt(jnp.sum(feat * feat, axis=1))))
    s1_ref[...] += jnp.concatenate(m1, axis=0)
    s2_ref[...] += jnp.concatenate(m2, axis=0)

    csum = jnp.sum(cw, axis=0, keepdims=True)
    row0 = jnp.concatenate([csum, jnp.zeros((1, D - K), F32)], axis=1)
    lane = lax.broadcasted_iota(jnp.int32, (1, D), 1)
    row1 = (jnp.where(lane == 0, nsum[0], 0.0)
            + jnp.where(lane == 1, nsum[1], 0.0)
            + jnp.where(lane == 2, nsum[2], 0.0)).astype(F32)
    aux_ref[...] += jnp.concatenate(
        [row0, row1, jnp.zeros((6, D), F32)], axis=0)


_prep_call = pl.pallas_call(
    _prep_body,
    grid=(NBA,),
    in_specs=[
        pl.BlockSpec((BRA, K), lambda i: (i, 0)),
        pl.BlockSpec((BRA, D), lambda i: (i, 0)),
        pl.BlockSpec((BRA, D), lambda i: (i, 0)),
        pl.BlockSpec((BRA, D), lambda i: (i, 0)),
        pl.BlockSpec((BRA, D), lambda i: (i, 0)),
    ],
    out_specs=[
        pl.BlockSpec((BRA, DA), lambda i: (i, 0)),
        pl.BlockSpec((3 * K, D), lambda i: (0, 0)),
        pl.BlockSpec((3 * K, D), lambda i: (0, 0)),
        pl.BlockSpec((8, D), lambda i: (0, 0)),
    ],
    out_shape=[
        jax.ShapeDtypeStruct((N, DA), BF16),
        jax.ShapeDtypeStruct((3 * K, D), F32),
        jax.ShapeDtypeStruct((3 * K, D), F32),
        jax.ShapeDtypeStruct((8, D), F32),
    ],
)


def _edge_body(row_hbm, col_hbm, table_hbm, zeros_hbm, out_hbm,
               row_v, col_v, rows0, rows1, acc_sh, gsem0, gsem1):
    c = lax.axis_index("c")
    s = lax.axis_index("s")
    wid = s * NC + c
    rps = NP // NS
    pltpu.sync_copy(zeros_hbm, acc_sh.at[pl.ds(s * rps, rps)])

    pltpu.sync_copy(row_hbm.at[pl.ds(wid * EW, EW)], row_v)
    pltpu.sync_copy(col_hbm.at[pl.ds(wid * EW, EW)], col_v)
    plsc.subcore_barrier()

    bufs = (rows0, rows1)
    gsems = (gsem0, gsem1)

    def cidx(ref, g):
        return ref.at[pl.ds(g * CH, CH)]

    def gather_start(g, b):
        pltpu.async_copy(table_hbm.at[cidx(col_v, g)], bufs[b], gsems[b])

    def gather_wait(g, b):
        pltpu.make_async_copy(table_hbm.at[cidx(col_v, g)], bufs[b],
                              gsems[b]).wait()

    def scatter(g, b):
        pltpu.sync_copy(bufs[b], acc_sh.at[cidx(row_v, g)], add=True)

    gather_start(0, 0)
    gather_start(1, 1)

    def step(i, carry):
        for b in range(2):
            g = 2 * i + b
            gather_wait(g, b)
            scatter(g, b)
            gather_start(g + 2, b)
        return carry

    lax.fori_loop(0, NCH // 2 - 1, step, 0)
    for b in range(2):
        gather_wait(NCH - 2 + b, b)
        scatter(NCH - 2 + b, b)

    plsc.subcore_barrier()
    pltpu.sync_copy(acc_sh.at[pl.ds(s * rps, rps)],
                    out_hbm.at[c, pl.ds(s * rps, rps)])


@functools.cache
def _edge_call():
    return functools.partial(
        pl.kernel,
        out_type=jax.ShapeDtypeStruct((NC, NP, DA), BF16),
        mesh=plsc.VectorSubcoreMesh(core_axis_name="c", subcore_axis_name="s",
                                    num_cores=NC, num_subcores=NS),
        scratch_types=[
            pltpu.VMEM((EW,), jnp.int32),
            pltpu.VMEM((EW,), jnp.int32),
            pltpu.VMEM((CH, DA), BF16),
            pltpu.VMEM((CH, DA), BF16),
            pltpu.VMEM_SHARED((NP, DA), BF16),
            pltpu.SemaphoreType.DMA,
            pltpu.SemaphoreType.DMA,
        ],
        compiler_params=pltpu.CompilerParams(use_tc_tiling_on_sc=False),
    )(_edge_body)


def _combine_body(parts_ref, table_ref, cw_ref, s1_ref, s2_ref, aux_ref,
                  out_ref, nacc_ref):
    i = pl.program_id(0)

    @pl.when(i == 0)
    def _():
        nacc_ref[...] = jnp.zeros_like(nacc_ref)

    p = parts_ref[...].astype(F32)
    ssum = p[0] + p[1]
    sv = ssum[:, :D]
    deg = ssum[:, D:D + 1] + 1e-8
    normed = table_ref[:, :D].astype(F32)
    nc = jnp.sum(normed * sv, axis=1, keepdims=True) / deg
    dn = (((0,), (0,)), ((), ()))
    nacc_ref[...] += lax.dot_general(nc, cw_ref[...], dn,
                                     preferred_element_type=F32)

    @pl.when(i == NB - 1)
    def _():
        s1 = s1_ref[...]
        s2 = s2_ref[...]
        var = (s2 - s1 * s1 * (1.0 / N)) * (1.0 / (N - 1))
        vmean = jnp.mean(var, axis=1, keepdims=True)
        w = jnp.concatenate([jnp.full((K, 1), 1.0, F32),
                             jnp.full((K, 1), 0.5, F32),
                             jnp.full((K, 1), 0.25, F32)], axis=0)
        hop_loss = jnp.sum(w / (vmean + 1e-8)) / K
        aux = aux_ref[...]
        csum = aux[0:1, :K]
        congestion = jnp.sum(nacc_ref[...] / (csum + 1e-8)) / K
        m0 = aux[1, 0] / N
        m1 = aux[1, 1] / N
        m2 = aux[1, 2] / N
        flow = jnp.maximum(m1 - m0, 0.0) + jnp.maximum(m2 - m1, 0.0)
        total = hop_loss + 0.5 * congestion + flow
        out_ref[...] = jnp.broadcast_to(total, (1, 1)).astype(F32)


_combine_call = pl.pallas_call(
    _combine_body,
    grid=(NB,),
    in_specs=[
        pl.BlockSpec((NC, BR, DA), lambda i: (0, i, 0)),
        pl.BlockSpec((BR, DA), lambda i: (i, 0)),
        pl.BlockSpec((BR, K), lambda i: (i, 0)),
        pl.BlockSpec((3 * K, D), lambda i: (0, 0)),
        pl.BlockSpec((3 * K, D), lambda i: (0, 0)),
        pl.BlockSpec((8, D), lambda i: (0, 0)),
    ],
    out_specs=pl.BlockSpec((1, 1), lambda i: (0, 0)),
    out_shape=jax.ShapeDtypeStruct((1, 1), F32),
    scratch_shapes=[pltpu.VMEM((1, K), F32)],
)


@jax.jit
def kernel(cluster_assignments, network_embeddings, hop_0_features,
           hop_1_features, hop_2_features, edge_index):
    table, s1, s2, aux = _prep_call(
        cluster_assignments, network_embeddings,
        hop_0_features, hop_1_features, hop_2_features)
    zeros = jnp.zeros((NP // NS, DA), BF16)
    parts = _edge_call()(edge_index[0], edge_index[1], table, zeros)
    total = _combine_call(parts, table, cluster_assignments, s1, s2, aux)
    return total[0, 0]


# speedup vs baseline: 1.6638x; 1.1126x over previous
import functools

import jax
import jax.numpy as jnp
from jax import lax
from jax.experimental import pallas as pl
from jax.experimental.pallas import tpu as pltpu
from jax.experimental.pallas import tpu_sc as plsc

N = 10000
K = 16
D = 128
DA = 160
E = 320000
NB = 10
BR = N // NB
NBA = 5
BRA = N // NBA
NC = 2
NS = 16
NW = NC * NS
EW = E // NW
CH = 200
NCH = EW // CH
NP = 10240
F32 = jnp.float32
BF16 = jnp.bfloat16


def _prep_body(cw_ref, emb_ref, h0_ref, h1_ref, h2_ref,
               table_ref, s1_ref, s2_ref, aux_ref):
    i = pl.program_id(0)
    cw = cw_ref[...]
    emb = emb_ref[...]
    nrm = jnp.sqrt(jnp.sum(emb * emb, axis=1, keepdims=True))
    normed = emb / jnp.maximum(nrm, 1e-8)
    table_ref[...] = jnp.concatenate(
        [normed, jnp.ones((BRA, 1), F32), jnp.zeros((BRA, DA - D - 1), F32)],
        axis=1).astype(BF16)

    @pl.when(i == 0)
    def _():
        s1_ref[...] = jnp.zeros_like(s1_ref)
        s2_ref[...] = jnp.zeros_like(s2_ref)
        aux_ref[...] = jnp.zeros_like(aux_ref)

    cw2 = cw * cw
    dn = (((0,), (0,)), ((), ()))
    m1 = []
    m2 = []
    nsum = []
    for f_ref in (h0_ref, h1_ref, h2_ref):
        feat = f_ref[...]
        m1.append(lax.dot_general(cw, feat, dn, preferred_element_type=F32))
        m2.append(lax.dot_general(cw2, feat * feat, dn,
                                  preferred_element_type=F32))
        nsum.append(jnp.sum(jnp.sqrt(jnp.sum(feat * feat, axis=1))))
    s1_ref[...] += jnp.concatenate(m1, axis=0)
    s2_ref[...] += jnp.concatenate(m2, axis=0)

    csum = jnp.sum(cw, axis=0, keepdims=True)
    row0 = jnp.concatenate([csum, jnp.zeros((1, D - K), F32)], axis=1)
    lane = lax.broadcasted_iota(jnp.int32, (1, D), 1)
    row1 = (jnp.where(lane == 0, nsum[0], 0.0)
            + jnp.where(lane == 1, nsum[1], 0.0)
            + jnp.where(lane == 2, nsum[2], 0.0)).astype(F32)
    aux_ref[...] += jnp.concatenate(
        [row0, row1, jnp.zeros((6, D), F32)], axis=0)


_prep_call = pl.pallas_call(
    _prep_body,
    grid=(NBA,),
    in_specs=[
        pl.BlockSpec((BRA, K), lambda i: (i, 0)),
        pl.BlockSpec((BRA, D), lambda i: (i, 0)),
        pl.BlockSpec((BRA, D), lambda i: (i, 0)),
        pl.BlockSpec((BRA, D), lambda i: (i, 0)),
        pl.BlockSpec((BRA, D), lambda i: (i, 0)),
    ],
    out_specs=[
        pl.BlockSpec((BRA, DA), lambda i: (i, 0)),
        pl.BlockSpec((3 * K, D), lambda i: (0, 0)),
        pl.BlockSpec((3 * K, D), lambda i: (0, 0)),
        pl.BlockSpec((8, D), lambda i: (0, 0)),
    ],
    out_shape=[
        jax.ShapeDtypeStruct((N, DA), BF16),
        jax.ShapeDtypeStruct((3 * K, D), F32),
        jax.ShapeDtypeStruct((3 * K, D), F32),
        jax.ShapeDtypeStruct((8, D), F32),
    ],
)


def _edge_body(row_hbm, col_hbm, table_hbm, zeros_hbm, out_hbm,
               row_v, col_v, rows0, rows1, ostage, acc_sh, gsem0, gsem1):
    c = lax.axis_index("c")
    s = lax.axis_index("s")
    wid = s * NC + c
    rps = NP // NS
    pltpu.sync_copy(zeros_hbm, acc_sh.at[pl.ds(s * rps, rps)])

    pltpu.sync_copy(row_hbm.at[pl.ds(wid * EW, EW)], row_v)
    pltpu.sync_copy(col_hbm.at[pl.ds(wid * EW, EW)], col_v)
    plsc.subcore_barrier()

    bufs = (rows0, rows1)
    gsems = (gsem0, gsem1)

    def cidx(ref, g):
        return ref.at[pl.ds(g * CH, CH)]

    def gather_start(g, b):
        pltpu.async_copy(table_hbm.at[cidx(col_v, g)], bufs[b], gsems[b])

    def gather_wait(g, b):
        pltpu.make_async_copy(table_hbm.at[cidx(col_v, g)], bufs[b],
                              gsems[b]).wait()

    def scatter(g, b):
        pltpu.sync_copy(bufs[b], acc_sh.at[cidx(row_v, g)], add=True)

    gather_start(0, 0)
    gather_start(1, 1)

    def step(i, carry):
        for b in range(2):
            g = 2 * i + b
            gather_wait(g, b)
            scatter(g, b)
            gather_start(g + 2, b)
        return carry

    lax.fori_loop(0, NCH // 2 - 1, step, 0)
    for b in range(2):
        gather_wait(NCH - 2 + b, b)
        scatter(NCH - 2 + b, b)

    plsc.subcore_barrier()

    PZ = 80

    def piece(p, carry):
        base = s * rps + p * PZ
        pltpu.sync_copy(acc_sh.at[pl.ds(base, PZ)], rows0.at[pl.ds(0, PZ)])
        pltpu.sync_copy(table_hbm.at[pl.ds(base, PZ)], rows1.at[pl.ds(0, PZ)])

        def rowfn(j, carry2):
            acc = jnp.zeros((16,), F32)
            for q in range(4):
                prod = (rows0[j, pl.ds(32 * q, 32)]
                        * rows1[j, pl.ds(32 * q, 32)])
                lo, hi = plsc.unpack(prod, format=plsc.PackFormat.INTERLEAVED)
                acc = acc + lo + hi
            dv = rows0[j, pl.ds(D, 32)]
            dlo, _ = plsc.unpack(dv, format=plsc.PackFormat.INTERLEAVED)
            ostage[p * PZ + j, :] = plsc.pack(
                acc, dlo, format=plsc.PackFormat.INTERLEAVED)
            return carry2

        lax.fori_loop(0, PZ, rowfn, 0)
        return carry

    @pl.when(s < NS - 1)
    def _():
        lax.fori_loop(0, rps // PZ, piece, 0)
        pltpu.sync_copy(ostage, out_hbm.at[c, pl.ds(s * rps, rps)])

    @pl.when(s == NS - 1)
    def _():
        nlast = (N - (NS - 1) * rps) // PZ
        lax.fori_loop(0, nlast, piece, 0)
        pltpu.sync_copy(ostage.at[pl.ds(0, nlast * PZ)],
                        out_hbm.at[c, pl.ds((NS - 1) * rps, nlast * PZ)])


@functools.cache
def _edge_call():
    return functools.partial(
        pl.kernel,
        out_type=jax.ShapeDtypeStruct((NC, NP, 32), BF16),
        mesh=plsc.VectorSubcoreMesh(core_axis_name="c", subcore_axis_name="s",
                                    num_cores=NC, num_subcores=NS),
        scratch_types=[
            pltpu.VMEM((EW,), jnp.int32),
            pltpu.VMEM((EW,), jnp.int32),
            pltpu.VMEM((CH, DA), BF16),
            pltpu.VMEM((CH, DA), BF16),
            pltpu.VMEM((NP // NS, 32), BF16),
            pltpu.VMEM_SHARED((NP, DA), BF16),
            pltpu.SemaphoreType.DMA,
            pltpu.SemaphoreType.DMA,
        ],
        compiler_params=pltpu.CompilerParams(use_tc_tiling_on_sc=False,
                                             needs_layout_passes=False),
    )(_edge_body)


def _combine_body(parts_ref, cw_ref, s1_ref, s2_ref, aux_ref,
                  out_ref, nacc_ref):
    i = pl.program_id(0)

    @pl.when(i == 0)
    def _():
        nacc_ref[...] = jnp.zeros_like(nacc_ref)

    p = parts_ref[...].astype(F32)
    ssum = p[0] + p[1]
    lane = lax.broadcasted_iota(jnp.int32, (BR, 32), 1)
    even = (lane % 2) == 0
    nc_raw = jnp.sum(jnp.where(even, ssum, 0.0), axis=1, keepdims=True)
    deg = jnp.sum(jnp.where(even, 0.0, ssum), axis=1, keepdims=True) + 1e-8
    nc = nc_raw / deg
    dn = (((0,), (0,)), ((), ()))
    nacc_ref[...] += lax.dot_general(nc, cw_ref[...], dn,
                                     preferred_element_type=F32)

    @pl.when(i == NB - 1)
    def _():
        s1 = s1_ref[...]
        s2 = s2_ref[...]
        var = (s2 - s1 * s1 * (1.0 / N)) * (1.0 / (N - 1))
        vmean = jnp.mean(var, axis=1, keepdims=True)
        w = jnp.concatenate([jnp.full((K, 1), 1.0, F32),
                             jnp.full((K, 1), 0.5, F32),
                             jnp.full((K, 1), 0.25, F32)], axis=0)
        hop_loss = jnp.sum(w / (vmean + 1e-8)) / K
        aux = aux_ref[...]
        csum = aux[0:1, :K]
        congestion = jnp.sum(nacc_ref[...] / (csum + 1e-8)) / K
        m0 = aux[1, 0] / N
        m1 = aux[1, 1] / N
        m2 = aux[1, 2] / N
        flow = jnp.maximum(m1 - m0, 0.0) + jnp.maximum(m2 - m1, 0.0)
        total = hop_loss + 0.5 * congestion + flow
        out_ref[...] = jnp.broadcast_to(total, (1, 1)).astype(F32)


_combine_call = pl.pallas_call(
    _combine_body,
    grid=(NB,),
    in_specs=[
        pl.BlockSpec((NC, BR, 32), lambda i: (0, i, 0)),
        pl.BlockSpec((BR, K), lambda i: (i, 0)),
        pl.BlockSpec((3 * K, D), lambda i: (0, 0)),
        pl.BlockSpec((3 * K, D), lambda i: (0, 0)),
        pl.BlockSpec((8, D), lambda i: (0, 0)),
    ],
    out_specs=pl.BlockSpec((1, 1), lambda i: (0, 0)),
    out_shape=jax.ShapeDtypeStruct((1, 1), F32),
    scratch_shapes=[pltpu.VMEM((1, K), F32)],
)


@jax.jit
def kernel(cluster_assignments, network_embeddings, hop_0_features,
           hop_1_features, hop_2_features, edge_index):
    table, s1, s2, aux = _prep_call(
        cluster_assignments, network_embeddings,
        hop_0_features, hop_1_features, hop_2_features)
    zeros = jnp.zeros((NP // NS, DA), BF16)
    parts = _edge_call()(edge_index[0], edge_index[1], table, zeros)
    total = _combine_call(parts, cluster_assignments, s1, s2, aux)
    return total[0, 0]


# speedup vs baseline: 1.6752x; 1.0069x over previous
import functools

import jax
import jax.numpy as jnp
from jax import lax
from jax.experimental import pallas as pl
from jax.experimental.pallas import tpu as pltpu
from jax.experimental.pallas import tpu_sc as plsc

N = 10000
K = 16
D = 128
DA = 160
E = 320000
NB = 5
BR = N // NB
NBA = 2
BRA = N // NBA
NC = 2
NS = 16
NW = NC * NS
EW = E // NW
CH = 200
NCH = EW // CH
NP = 10240
F32 = jnp.float32
BF16 = jnp.bfloat16


def _prep_body(cw_ref, emb_ref, h0_ref, h1_ref, h2_ref,
               table_ref, s1_ref, s2_ref, aux_ref):
    i = pl.program_id(0)
    cw = cw_ref[...]
    emb = emb_ref[...]
    nrm = jnp.sqrt(jnp.sum(emb * emb, axis=1, keepdims=True))
    normed = emb / jnp.maximum(nrm, 1e-8)
    table_ref[...] = jnp.concatenate(
        [normed, jnp.ones((BRA, 1), F32), jnp.zeros((BRA, DA - D - 1), F32)],
        axis=1).astype(BF16)

    @pl.when(i == 0)
    def _():
        s1_ref[...] = jnp.zeros_like(s1_ref)
        s2_ref[...] = jnp.zeros_like(s2_ref)
        aux_ref[...] = jnp.zeros_like(aux_ref)

    cw2 = cw * cw
    dn = (((0,), (0,)), ((), ()))
    m1 = []
    m2 = []
    nsum = []
    for f_ref in (h0_ref, h1_ref, h2_ref):
        feat = f_ref[...]
        m1.append(lax.dot_general(cw, feat, dn, preferred_element_type=F32))
        m2.append(lax.dot_general(cw2, feat * feat, dn,
                                  preferred_element_type=F32))
        nsum.append(jnp.sum(jnp.sqrt(jnp.sum(feat * feat, axis=1))))
    s1_ref[...] += jnp.concatenate(m1, axis=0)
    s2_ref[...] += jnp.concatenate(m2, axis=0)

    csum = jnp.sum(cw, axis=0, keepdims=True)
    row0 = jnp.concatenate([csum, jnp.zeros((1, D - K), F32)], axis=1)
    lane = lax.broadcasted_iota(jnp.int32, (1, D), 1)
    row1 = (jnp.where(lane == 0, nsum[0], 0.0)
            + jnp.where(lane == 1, nsum[1], 0.0)
            + jnp.where(lane == 2, nsum[2], 0.0)).astype(F32)
    aux_ref[...] += jnp.concatenate(
        [row0, row1, jnp.zeros((6, D), F32)], axis=0)


_prep_call = pl.pallas_call(
    _prep_body,
    grid=(NBA,),
    in_specs=[
        pl.BlockSpec((BRA, K), lambda i: (i, 0)),
        pl.BlockSpec((BRA, D), lambda i: (i, 0)),
        pl.BlockSpec((BRA, D), lambda i: (i, 0)),
        pl.BlockSpec((BRA, D), lambda i: (i, 0)),
        pl.BlockSpec((BRA, D), lambda i: (i, 0)),
    ],
    out_specs=[
        pl.BlockSpec((BRA, DA), lambda i: (i, 0)),
        pl.BlockSpec((3 * K, D), lambda i: (0, 0)),
        pl.BlockSpec((3 * K, D), lambda i: (0, 0)),
        pl.BlockSpec((8, D), lambda i: (0, 0)),
    ],
    out_shape=[
        jax.ShapeDtypeStruct((N, DA), BF16),
        jax.ShapeDtypeStruct((3 * K, D), F32),
        jax.ShapeDtypeStruct((3 * K, D), F32),
        jax.ShapeDtypeStruct((8, D), F32),
    ],
)


def _edge_body(row_hbm, col_hbm, table_hbm, zeros_hbm, out_hbm,
               row_v, col_v, rows0, rows1, ostage, acc_sh, gsem0, gsem1):
    c = lax.axis_index("c")
    s = lax.axis_index("s")
    wid = s * NC + c
    rps = NP // NS
    pltpu.sync_copy(zeros_hbm, acc_sh.at[pl.ds(s * rps, rps)])

    pltpu.sync_copy(row_hbm.at[pl.ds(wid * EW, EW)], row_v)
    pltpu.sync_copy(col_hbm.at[pl.ds(wid * EW, EW)], col_v)
    plsc.subcore_barrier()

    bufs = (rows0, rows1)
    gsems = (gsem0, gsem1)

    def cidx(ref, g):
        return ref.at[pl.ds(g * CH, CH)]

    def gather_start(g, b):
        pltpu.async_copy(table_hbm.at[cidx(col_v, g)], bufs[b], gsems[b])

    def gather_wait(g, b):
        pltpu.make_async_copy(table_hbm.at[cidx(col_v, g)], bufs[b],
                              gsems[b]).wait()

    def scatter(g, b):
        pltpu.sync_copy(bufs[b], acc_sh.at[cidx(row_v, g)], add=True)

    gather_start(0, 0)
    gather_start(1, 1)

    def step(i, carry):
        for b in range(2):
            g = 2 * i + b
            gather_wait(g, b)
            scatter(g, b)
            gather_start(g + 2, b)
        return carry

    lax.fori_loop(0, NCH // 2 - 1, step, 0)
    for b in range(2):
        gather_wait(NCH - 2 + b, b)
        scatter(NCH - 2 + b, b)

    plsc.subcore_barrier()

    PZ = 80

    def piece(p, carry):
        base = s * rps + p * PZ
        pltpu.sync_copy(acc_sh.at[pl.ds(base, PZ)], rows0.at[pl.ds(0, PZ)])
        pltpu.sync_copy(table_hbm.at[pl.ds(base, PZ)], rows1.at[pl.ds(0, PZ)])

        def rowfn(j, carry2):
            acc = jnp.zeros((16,), F32)
            for q in range(4):
                prod = (rows0[j, pl.ds(32 * q, 32)]
                        * rows1[j, pl.ds(32 * q, 32)])
                lo, hi = plsc.unpack(prod, format=plsc.PackFormat.INTERLEAVED)
                acc = acc + lo + hi
            dv = rows0[j, pl.ds(D, 32)]
            dlo, _ = plsc.unpack(dv, format=plsc.PackFormat.INTERLEAVED)
            ostage[p * PZ + j, :] = plsc.pack(
                acc, dlo, format=plsc.PackFormat.INTERLEAVED)
            return carry2

        lax.fori_loop(0, PZ, rowfn, 0)
        return carry

    @pl.when(s < NS - 1)
    def _():
        lax.fori_loop(0, rps // PZ, piece, 0)
        pltpu.sync_copy(ostage, out_hbm.at[c, pl.ds(s * rps, rps)])

    @pl.when(s == NS - 1)
    def _():
        nlast = (N - (NS - 1) * rps) // PZ
        lax.fori_loop(0, nlast, piece, 0)
        pltpu.sync_copy(ostage.at[pl.ds(0, nlast * PZ)],
                        out_hbm.at[c, pl.ds((NS - 1) * rps, nlast * PZ)])


@functools.cache
def _edge_call():
    return functools.partial(
        pl.kernel,
        out_type=jax.ShapeDtypeStruct((NC, NP, 32), BF16),
        mesh=plsc.VectorSubcoreMesh(core_axis_name="c", subcore_axis_name="s",
                                    num_cores=NC, num_subcores=NS),
        scratch_types=[
            pltpu.VMEM((EW,), jnp.int32),
            pltpu.VMEM((EW,), jnp.int32),
            pltpu.VMEM((CH, DA), BF16),
            pltpu.VMEM((CH, DA), BF16),
            pltpu.VMEM((NP // NS, 32), BF16),
            pltpu.VMEM_SHARED((NP, DA), BF16),
            pltpu.SemaphoreType.DMA,
            pltpu.SemaphoreType.DMA,
        ],
        compiler_params=pltpu.CompilerParams(use_tc_tiling_on_sc=False,
                                             needs_layout_passes=False),
    )(_edge_body)


def _combine_body(parts_ref, cw_ref, s1_ref, s2_ref, aux_ref,
                  out_ref, nacc_ref):
    i = pl.program_id(0)

    @pl.when(i == 0)
    def _():
        nacc_ref[...] = jnp.zeros_like(nacc_ref)

    p = parts_ref[...].astype(F32)
    ssum = p[0] + p[1]
    lane = lax.broadcasted_iota(jnp.int32, (BR, 32), 1)
    even = (lane % 2) == 0
    nc_raw = jnp.sum(jnp.where(even, ssum, 0.0), axis=1, keepdims=True)
    deg = jnp.sum(jnp.where(even, 0.0, ssum), axis=1, keepdims=True) + 1e-8
    nc = nc_raw / deg
    dn = (((0,), (0,)), ((), ()))
    nacc_ref[...] += lax.dot_general(nc, cw_ref[...], dn,
                                     preferred_element_type=F32)

    @pl.when(i == NB - 1)
    def _():
        s1 = s1_ref[...]
        s2 = s2_ref[...]
        var = (s2 - s1 * s1 * (1.0 / N)) * (1.0 / (N - 1))
        vmean = jnp.mean(var, axis=1, keepdims=True)
        w = jnp.concatenate([jnp.full((K, 1), 1.0, F32),
                             jnp.full((K, 1), 0.5, F32),
                             jnp.full((K, 1), 0.25, F32)], axis=0)
        hop_loss = jnp.sum(w / (vmean + 1e-8)) / K
        aux = aux_ref[...]
        csum = aux[0:1, :K]
        congestion = jnp.sum(nacc_ref[...] / (csum + 1e-8)) / K
        m0 = aux[1, 0] / N
        m1 = aux[1, 1] / N
        m2 = aux[1, 2] / N
        flow = jnp.maximum(m1 - m0, 0.0) + jnp.maximum(m2 - m1, 0.0)
        total = hop_loss + 0.5 * congestion + flow
        out_ref[...] = jnp.broadcast_to(total, (1, 1)).astype(F32)


_combine_call = pl.pallas_call(
    _combine_body,
    grid=(NB,),
    in_specs=[
        pl.BlockSpec((NC, BR, 32), lambda i: (0, i, 0)),
        pl.BlockSpec((BR, K), lambda i: (i, 0)),
        pl.BlockSpec((3 * K, D), lambda i: (0, 0)),
        pl.BlockSpec((3 * K, D), lambda i: (0, 0)),
        pl.BlockSpec((8, D), lambda i: (0, 0)),
    ],
    out_specs=pl.BlockSpec((1, 1), lambda i: (0, 0)),
    out_shape=jax.ShapeDtypeStruct((1, 1), F32),
    scratch_shapes=[pltpu.VMEM((1, K), F32)],
)


@jax.jit
def kernel(cluster_assignments, network_embeddings, hop_0_features,
           hop_1_features, hop_2_features, edge_index):
    table, s1, s2, aux = _prep_call(
        cluster_assignments, network_embeddings,
        hop_0_features, hop_1_features, hop_2_features)
    zeros = jnp.zeros((NP // NS, DA), BF16)
    parts = _edge_call()(edge_index[0], edge_index[1], table, zeros)
    total = _combine_call(parts, cluster_assignments, s1, s2, aux)
    return total[0, 0]


# speedup vs baseline: 1.7593x; 1.0502x over previous
import functools

import jax
import jax.numpy as jnp
from jax import lax
from jax.experimental import pallas as pl
from jax.experimental.pallas import tpu as pltpu
from jax.experimental.pallas import tpu_sc as plsc

N = 10000
K = 16
D = 128
DA = 160
E = 320000
NB = 5
BR = N // NB
NBA = 2
BRA = N // NBA
NC = 2
NS = 16
NW = NC * NS
EW = E // NW
CH = 200
NCH = EW // CH
NP = 10240
F32 = jnp.float32
BF16 = jnp.bfloat16


def _prep_body(cw_ref, emb_ref, h0_ref, h1_ref, h2_ref,
               table_ref, s1_ref, s2_ref, aux_ref):
    i = pl.program_id(0)
    cw = cw_ref[...]
    emb = emb_ref[...]
    nrm = jnp.sqrt(jnp.sum(emb * emb, axis=1, keepdims=True))
    normed = emb / jnp.maximum(nrm, 1e-8)
    table_ref[...] = jnp.concatenate(
        [normed, jnp.ones((BRA, 1), F32), jnp.zeros((BRA, DA - D - 1), F32)],
        axis=1).astype(BF16)

    @pl.when(i == 0)
    def _():
        s1_ref[...] = jnp.zeros_like(s1_ref)
        s2_ref[...] = jnp.zeros_like(s2_ref)
        aux_ref[...] = jnp.zeros_like(aux_ref)

    cw2 = cw * cw
    dn = (((0,), (0,)), ((), ()))
    m1 = []
    m2 = []
    nsum = []
    for f_ref in (h0_ref, h1_ref, h2_ref):
        feat = f_ref[...]
        m1.append(lax.dot_general(cw, feat, dn, preferred_element_type=F32))
        m2.append(lax.dot_general(cw2, feat * feat, dn,
                                  preferred_element_type=F32))
        nsum.append(jnp.sum(jnp.sqrt(jnp.sum(feat * feat, axis=1))))
    s1_ref[...] += jnp.concatenate(m1, axis=0)
    s2_ref[...] += jnp.concatenate(m2, axis=0)

    csum = jnp.sum(cw, axis=0, keepdims=True)
    row0 = jnp.concatenate([csum, jnp.zeros((1, D - K), F32)], axis=1)
    lane = lax.broadcasted_iota(jnp.int32, (1, D), 1)
    row1 = (jnp.where(lane == 0, nsum[0], 0.0)
            + jnp.where(lane == 1, nsum[1], 0.0)
            + jnp.where(lane == 2, nsum[2], 0.0)).astype(F32)
    aux_ref[...] += jnp.concatenate(
        [row0, row1, jnp.zeros((6, D), F32)], axis=0)


_prep_call = pl.pallas_call(
    _prep_body,
    grid=(NBA,),
    in_specs=[
        pl.BlockSpec((BRA, K), lambda i: (i, 0)),
        pl.BlockSpec((BRA, D), lambda i: (i, 0)),
        pl.BlockSpec((BRA, D), lambda i: (i, 0)),
        pl.BlockSpec((BRA, D), lambda i: (i, 0)),
        pl.BlockSpec((BRA, D), lambda i: (i, 0)),
    ],
    out_specs=[
        pl.BlockSpec((BRA, DA), lambda i: (i, 0)),
        pl.BlockSpec((3 * K, D), lambda i: (0, 0)),
        pl.BlockSpec((3 * K, D), lambda i: (0, 0)),
        pl.BlockSpec((8, D), lambda i: (0, 0)),
    ],
    out_shape=[
        jax.ShapeDtypeStruct((N, DA), BF16),
        jax.ShapeDtypeStruct((3 * K, D), F32),
        jax.ShapeDtypeStruct((3 * K, D), F32),
        jax.ShapeDtypeStruct((8, D), F32),
    ],
)


def _edge_body(row_hbm, col_hbm, table_hbm, zeros_hbm, out_hbm,
               row_v, col_v, rows0, rows1, ostage, acc_sh, gsem0, gsem1):
    c = lax.axis_index("c")
    s = lax.axis_index("s")
    wid = s * NC + c
    rps = NP // NS
    pltpu.sync_copy(zeros_hbm, acc_sh.at[pl.ds(s * rps, rps)])

    pltpu.sync_copy(row_hbm.at[pl.ds(wid * EW, EW)], row_v)
    pltpu.sync_copy(col_hbm.at[pl.ds(wid * EW, EW)], col_v)
    plsc.subcore_barrier()

    bufs = (rows0, rows1)
    gsems = (gsem0, gsem1)

    def cidx(ref, g):
        return ref.at[pl.ds(g * CH, CH)]

    def gather_start(g, b):
        pltpu.async_copy(table_hbm.at[cidx(col_v, g)], bufs[b], gsems[b])

    def gather_wait(g, b):
        pltpu.make_async_copy(table_hbm.at[cidx(col_v, g)], bufs[b],
                              gsems[b]).wait()

    def scatter(g, b):
        pltpu.sync_copy(bufs[b], acc_sh.at[cidx(row_v, g)], add=True)

    gather_start(0, 0)
    gather_start(1, 1)

    def step(i, carry):
        for b in range(2):
            g = 2 * i + b
            gather_wait(g, b)
            scatter(g, b)
            gather_start(g + 2, b)
        return carry

    lax.fori_loop(0, NCH // 2 - 1, step, 0)
    for b in range(2):
        gather_wait(NCH - 2 + b, b)
        scatter(NCH - 2 + b, b)

    plsc.subcore_barrier()

    PZ = 80

    def piece_fetch(p, b):
        base = s * rps + p * PZ
        pltpu.async_copy(acc_sh.at[pl.ds(base, PZ)],
                         rows0.at[pl.ds(b * PZ, PZ)], gsem0)
        pltpu.async_copy(table_hbm.at[pl.ds(base, PZ)],
                         rows1.at[pl.ds(b * PZ, PZ)], gsem1)

    def piece_wait(p, b):
        base = s * rps + p * PZ
        pltpu.make_async_copy(acc_sh.at[pl.ds(base, PZ)],
                              rows0.at[pl.ds(b * PZ, PZ)], gsem0).wait()
        pltpu.make_async_copy(table_hbm.at[pl.ds(base, PZ)],
                              rows1.at[pl.ds(b * PZ, PZ)], gsem1).wait()

    def piece_compute(p, b):
        def rowfn(j, carry2):
            acc = jnp.zeros((16,), F32)
            for q in range(4):
                prod = (rows0[b * PZ + j, pl.ds(32 * q, 32)]
                        * rows1[b * PZ + j, pl.ds(32 * q, 32)])
                lo, hi = plsc.unpack(prod, format=plsc.PackFormat.INTERLEAVED)
                acc = acc + lo + hi
            dv = rows0[b * PZ + j, pl.ds(D, 32)]
            dlo, _ = plsc.unpack(dv, format=plsc.PackFormat.INTERLEAVED)
            ostage[p * PZ + j, :] = plsc.pack(
                acc, dlo, format=plsc.PackFormat.INTERLEAVED)
            return carry2

        lax.fori_loop(0, PZ, rowfn, 0)

    def finalize(npieces):
        piece_fetch(0, 0)
        piece_fetch(1, 1)

        def fstep(i, carry):
            for b in range(2):
                p = 2 * i + b
                piece_wait(p, b)
                piece_compute(p, b)
                pl.when(p + 2 < npieces)(lambda: piece_fetch(p + 2, b))
            return carry

        lax.fori_loop(0, npieces // 2, fstep, 0)

    @pl.when(s < NS - 1)
    def _():
        finalize(rps // PZ)
        pltpu.sync_copy(ostage, out_hbm.at[c, pl.ds(s * rps, rps)])

    @pl.when(s == NS - 1)
    def _():
        nlast = (N - (NS - 1) * rps) // PZ
        piece_fetch(0, 0)
        for p in range(nlast):
            b = p % 2
            piece_wait(p, b)
            if p + 1 < nlast:
                piece_fetch(p + 1, (p + 1) % 2)
            piece_compute(p, b)
        pltpu.sync_copy(ostage.at[pl.ds(0, nlast * PZ)],
                        out_hbm.at[c, pl.ds((NS - 1) * rps, nlast * PZ)])


@functools.cache
def _edge_call():
    return functools.partial(
        pl.kernel,
        out_type=jax.ShapeDtypeStruct((NC, NP, 32), BF16),
        mesh=plsc.VectorSubcoreMesh(core_axis_name="c", subcore_axis_name="s",
                                    num_cores=NC, num_subcores=NS),
        scratch_types=[
            pltpu.VMEM((EW,), jnp.int32),
            pltpu.VMEM((EW,), jnp.int32),
            pltpu.VMEM((CH, DA), BF16),
            pltpu.VMEM((CH, DA), BF16),
            pltpu.VMEM((NP // NS, 32), BF16),
            pltpu.VMEM_SHARED((NP, DA), BF16),
            pltpu.SemaphoreType.DMA,
            pltpu.SemaphoreType.DMA,
        ],
        compiler_params=pltpu.CompilerParams(use_tc_tiling_on_sc=False,
                                             needs_layout_passes=False),
    )(_edge_body)


def _combine_body(parts_ref, cw_ref, s1_ref, s2_ref, aux_ref,
                  out_ref, nacc_ref):
    i = pl.program_id(0)

    @pl.when(i == 0)
    def _():
        nacc_ref[...] = jnp.zeros_like(nacc_ref)

    p = parts_ref[...].astype(F32)
    ssum = p[0] + p[1]
    lane = lax.broadcasted_iota(jnp.int32, (BR, 32), 1)
    even = (lane % 2) == 0
    nc_raw = jnp.sum(jnp.where(even, ssum, 0.0), axis=1, keepdims=True)
    deg = jnp.sum(jnp.where(even, 0.0, ssum), axis=1, keepdims=True) + 1e-8
    nc = nc_raw / deg
    dn = (((0,), (0,)), ((), ()))
    nacc_ref[...] += lax.dot_general(nc, cw_ref[...], dn,
                                     preferred_element_type=F32)

    @pl.when(i == NB - 1)
    def _():
        s1 = s1_ref[...]
        s2 = s2_ref[...]
        var = (s2 - s1 * s1 * (1.0 / N)) * (1.0 / (N - 1))
        vmean = jnp.mean(var, axis=1, keepdims=True)
        w = jnp.concatenate([jnp.full((K, 1), 1.0, F32),
                             jnp.full((K, 1), 0.5, F32),
                             jnp.full((K, 1), 0.25, F32)], axis=0)
        hop_loss = jnp.sum(w / (vmean + 1e-8)) / K
        aux = aux_ref[...]
        csum = aux[0:1, :K]
        congestion = jnp.sum(nacc_ref[...] / (csum + 1e-8)) / K
        m0 = aux[1, 0] / N
        m1 = aux[1, 1] / N
        m2 = aux[1, 2] / N
        flow = jnp.maximum(m1 - m0, 0.0) + jnp.maximum(m2 - m1, 0.0)
        total = hop_loss + 0.5 * congestion + flow
        out_ref[...] = jnp.broadcast_to(total, (1, 1)).astype(F32)


_combine_call = pl.pallas_call(
    _combine_body,
    grid=(NB,),
    in_specs=[
        pl.BlockSpec((NC, BR, 32), lambda i: (0, i, 0)),
        pl.BlockSpec((BR, K), lambda i: (i, 0)),
        pl.BlockSpec((3 * K, D), lambda i: (0, 0)),
        pl.BlockSpec((3 * K, D), lambda i: (0, 0)),
        pl.BlockSpec((8, D), lambda i: (0, 0)),
    ],
    out_specs=pl.BlockSpec((1, 1), lambda i: (0, 0)),
    out_shape=jax.ShapeDtypeStruct((1, 1), F32),
    scratch_shapes=[pltpu.VMEM((1, K), F32)],
)


@jax.jit
def kernel(cluster_assignments, network_embeddings, hop_0_features,
           hop_1_features, hop_2_features, edge_index):
    table, s1, s2, aux = _prep_call(
        cluster_assignments, network_embeddings,
        hop_0_features, hop_1_features, hop_2_features)
    zeros = jnp.zeros((NP // NS, DA), BF16)
    parts = _edge_call()(edge_index[0], edge_index[1], table, zeros)
    total = _combine_call(parts, cluster_assignments, s1, s2, aux)
    return total[0, 0]
